# baseline (speedup 1.0000x reference)
import functools

import numpy as np
import jax
import jax.numpy as jnp
from jax import lax
from jax.experimental import pallas as pl
from jax.experimental.pallas import tpu as pltpu
from jax.experimental.pallas import tpu_sc as plsc

_FIELD_DIMS = [100000] * 26
_F = 26
_D = 16
_B = 16384
_N = _B * _F
_NC, _NS = 2, 16
_NW = _NC * _NS
_PER_W = _N // _NW
_CHROWS = 8
_CH = _CHROWS * 128
_NCH = _PER_W // _CH
_ROWS_PER_W = _PER_W // 128
_PERIOD = 208

_OFF_PATTERN = np.tile(
    np.array((0, *np.cumsum(_FIELD_DIMS)[:-1]), dtype=np.int32),
    (_PERIOD + _CH + 16) // _F + 1)[: _PERIOD + _CH + 16]


def _emb_body(x_hbm, offs_hbm, table_hbm, out_hbm, xraw_v, offs_v, rows_v, sem):
    wid = lax.axis_index("s") * _NC + lax.axis_index("c")
    row0 = wid * _ROWS_PER_W
    pltpu.sync_copy(offs_hbm, offs_v)

    def chunk(c, carry):
        r0 = row0 + c * _CHROWS
        phase = lax.rem(c * _CH, _PERIOD)
        pltpu.sync_copy(x_hbm.at[pl.ds(r0, _CHROWS)], xraw_v)
        for j in range(_CHROWS):
            for k in range(128 // 16):
                sl = pl.ds(k * 16, 16)
                xraw_v[j, sl] = xraw_v[j, sl] + offs_v[
                    pl.ds(phase + j * 128 + k * 16, 16)]
        copies = [
            pltpu.async_copy(table_hbm.at[xraw_v.at[j]], rows_v.at[j], sem)
            for j in range(_CHROWS)
        ]
        for cp in copies:
            cp.wait()
        pltpu.sync_copy(rows_v, out_hbm.at[pl.ds(r0, _CHROWS)])
        return carry

    lax.fori_loop(0, _NCH, chunk, 0)


@jax.jit
def _emb_call(x2d, offs, table):
    mesh = plsc.VectorSubcoreMesh(core_axis_name="c", subcore_axis_name="s")
    f = pl.kernel(
        _emb_body,
        out_type=jax.ShapeDtypeStruct((_N // 128, 128, _D), jnp.float32),
        mesh=mesh,
        scratch_types=[
            pltpu.VMEM((_CHROWS, 128), jnp.int32),
            pltpu.VMEM((len(_OFF_PATTERN),), jnp.int32),
            pltpu.VMEM((_CHROWS, 128, _D), jnp.float32),
            pltpu.SemaphoreType.DMA,
        ],
        compiler_params=pltpu.CompilerParams(use_tc_tiling_on_sc=False),
    )
    return f(x2d, offs, table)


def kernel(x, table):
    x2d = x.astype(jnp.int32).reshape(_N // 128, 128)
    offs = jnp.asarray(_OFF_PATTERN)
    out = _emb_call(x2d, offs, table)
    return out.reshape(_B, _F, _D)
